# baseline (speedup 1.0000x reference)
import functools

import jax
import jax.numpy as jnp
from jax import lax
from jax.experimental import pallas as pl
from jax.experimental.pallas import tpu as pltpu
from jax.experimental.pallas import tpu_sc as plsc

N = 32768
D = 512
S = 16
NC = 2
NS = 16
L = 16
NW = NC * NS
RW = N // NW
CH = 32
NCHUNK = RW // CH
NBUF = 4
DV = D // L


def _sc_partial_sums(features, ids32):
    mesh = plsc.VectorSubcoreMesh(
        core_axis_name="c", subcore_axis_name="s", num_cores=NC, num_subcores=NS
    )

    @functools.partial(
        pl.kernel,
        out_type=(
            jax.ShapeDtypeStruct((NW, S, D), jnp.float32),
            jax.ShapeDtypeStruct((NW, S), jnp.float32),
        ),
        mesh=mesh,
        scratch_types=[
            pltpu.VMEM((RW,), jnp.int32),
            pltpu.VMEM((NBUF, CH, D), jnp.float32),
            pltpu.VMEM((S, D), jnp.float32),
            pltpu.VMEM((S,), jnp.float32),
        ] + [pltpu.SemaphoreType.DMA] * NBUF,
        compiler_params=pltpu.CompilerParams(needs_layout_passes=False),
    )
    def k(feat_hbm, ids_hbm, sums_hbm, cnts_hbm, ids_v, buf_v, part_v, cnt_v,
          *sems):
        wid = lax.axis_index("s") * NC + lax.axis_index("c")
        base = wid * RW

        pltpu.sync_copy(ids_hbm.at[pl.ds(base, RW)], ids_v)

        lane = lax.iota(jnp.int32, L)
        zero = jnp.zeros((L,), jnp.float32)

        def zbody(s, _):
            for j in range(DV):
                part_v[s, pl.ds(j * L, L)] = zero
            return 0
        lax.fori_loop(0, S, zbody, 0)

        def lower_bound(tgt):
            def step(_, lh):
                lo, hi = lh
                mid = lo + lax.shift_right_logical(hi - lo, 1)
                v = plsc.load_gather(ids_v, [jnp.minimum(mid, RW - 1)])
                go = mid < hi
                p = go & (v < tgt)
                lo = jnp.where(p, mid + 1, lo)
                hi = jnp.where(go & jnp.logical_not(p), mid, hi)
                return lo, hi
            lo0 = jnp.zeros((L,), jnp.int32)
            hi0 = jnp.full((L,), RW, jnp.int32)
            lo, _ = lax.fori_loop(0, 11, step, (lo0, hi0))
            return lo

        lb = lower_bound(lane)
        ub = lower_bound(lane + 1)

        cnt_v[...] = (ub - lb).astype(jnp.float32)
        pltpu.sync_copy(cnt_v, cnts_hbm.at[wid])

        def chunk_dma(c, b):
            return pltpu.async_copy(
                feat_hbm.at[pl.ds(base + c * CH, CH)], buf_v.at[b], sems[b]
            )

        for b in range(NBUF):
            chunk_dma(b, b)

        def chunk_body(g, b):
            c = g * NBUF + b
            pltpu.make_async_copy(
                feat_hbm.at[pl.ds(base + c * CH, CH)], buf_v.at[b], sems[b]
            ).wait()
            off = c * CH

            s_first = jnp.max(plsc.load_gather(
                ids_v, [jnp.full((L,), off, jnp.int32)]))
            s_last = jnp.max(plsc.load_gather(
                ids_v, [jnp.full((L,), off + CH - 1, jnp.int32)]))

            def seg_body(s, _):
                st = jnp.max(jnp.where(lane == s, lb, 0))
                en = jnp.max(jnp.where(lane == s, ub, 0))
                lo = jnp.clip(st - off, 0, CH)
                hi = jnp.clip(en - off, 0, CH)

                @pl.when(hi > lo)
                def _():
                    n = hi - lo
                    quads = lax.shift_right_logical(n, 2)

                    def row4(i, acc):
                        r = lo + i * 4
                        out = []
                        for j in range(DV):
                            sl = pl.ds(j * L, L)
                            x01 = buf_v[b, r, sl] + buf_v[b, r + 1, sl]
                            x23 = buf_v[b, r + 2, sl] + buf_v[b, r + 3, sl]
                            out.append(acc[j] + (x01 + x23))
                        return out

                    acc = lax.fori_loop(0, quads, row4, [zero] * DV)

                    def row(r, acc):
                        return [acc[j] + buf_v[b, r, pl.ds(j * L, L)]
                                for j in range(DV)]
                    acc = lax.fori_loop(lo + quads * 4, hi, row, acc)

                    for j in range(DV):
                        sl = pl.ds(j * L, L)
                        part_v[s, sl] = part_v[s, sl] + acc[j]
                return 0

            lax.fori_loop(s_first, s_last + 1, seg_body, 0)

            @pl.when(c + NBUF < NCHUNK)
            def _():
                chunk_dma(c + NBUF, b)

        def outer(g, _):
            for b in range(NBUF):
                chunk_body(g, b)
            return 0

        lax.fori_loop(0, NCHUNK // NBUF, outer, 0)

        pltpu.sync_copy(part_v, sums_hbm.at[wid])

    return k(features, ids32)


def _combine(psums, pcnts):
    def body(ps_ref, pc_ref, out_ref):
        sums = jnp.sum(ps_ref[...], axis=0)
        cnts = jnp.sum(pc_ref[...], axis=0)
        out_ref[...] = sums / jnp.maximum(cnts, 1.0)[:, None]

    return pl.pallas_call(
        body,
        out_shape=jax.ShapeDtypeStruct((S, D), jnp.float32),
    )(psums, pcnts)


@jax.jit
def kernel(features, batch_ids):
    ids32 = batch_ids.astype(jnp.int32)
    psums, pcnts = _sc_partial_sums(features, ids32)
    return _combine(psums, pcnts)


# speedup vs baseline: 1.4620x; 1.4620x over previous
import functools

import jax
import jax.numpy as jnp
from jax import lax
from jax.experimental import pallas as pl
from jax.experimental.pallas import tpu as pltpu
from jax.experimental.pallas import tpu_sc as plsc

N = 32768
D = 512
S = 16
NC = 2
NS = 16
L = 16
NW = NC * NS

NSC = 16384
RW = NSC // NW
CH = 64
NCHUNK = RW // CH
NBUF = 2
DV = D // L

BR = 2048
NTCBLK = (N - NSC) // BR


def _sc_partial_sums(features, ids32):
    mesh = plsc.VectorSubcoreMesh(
        core_axis_name="c", subcore_axis_name="s", num_cores=NC, num_subcores=NS
    )

    @functools.partial(
        pl.kernel,
        out_type=(
            jax.ShapeDtypeStruct((NW, S, D), jnp.float32),
            jax.ShapeDtypeStruct((NW, S), jnp.float32),
        ),
        mesh=mesh,
        scratch_types=[
            pltpu.VMEM((RW,), jnp.int32),
            pltpu.VMEM((NBUF, CH, D), jnp.float32),
            pltpu.VMEM((S, D), jnp.float32),
            pltpu.VMEM((S,), jnp.float32),
        ] + [pltpu.SemaphoreType.DMA] * NBUF,
        compiler_params=pltpu.CompilerParams(needs_layout_passes=False),
    )
    def k(feat_hbm, ids_hbm, sums_hbm, cnts_hbm, ids_v, buf_v, part_v, cnt_v,
          *sems):
        wid = lax.axis_index("s") * NC + lax.axis_index("c")
        base = wid * RW

        pltpu.sync_copy(ids_hbm.at[pl.ds(base, RW)], ids_v)

        lane = lax.iota(jnp.int32, L)
        zero = jnp.zeros((L,), jnp.float32)

        def zbody(s, _):
            for j in range(DV):
                part_v[s, pl.ds(j * L, L)] = zero
            return 0
        lax.fori_loop(0, S, zbody, 0)

        def lower_bound(tgt):
            def step(_, lh):
                lo, hi = lh
                mid = lo + lax.shift_right_logical(hi - lo, 1)
                v = plsc.load_gather(ids_v, [jnp.minimum(mid, RW - 1)])
                go = mid < hi
                p = go & (v < tgt)
                lo = jnp.where(p, mid + 1, lo)
                hi = jnp.where(go & jnp.logical_not(p), mid, hi)
                return lo, hi
            lo0 = jnp.zeros((L,), jnp.int32)
            hi0 = jnp.full((L,), RW, jnp.int32)
            lo, _ = lax.fori_loop(0, 11, step, (lo0, hi0))
            return lo

        lb = lower_bound(lane)
        ub = lower_bound(lane + 1)

        cnt_v[...] = (ub - lb).astype(jnp.float32)
        pltpu.sync_copy(cnt_v, cnts_hbm.at[wid])

        def chunk_dma(c, b):
            return pltpu.async_copy(
                feat_hbm.at[pl.ds(base + c * CH, CH)], buf_v.at[b], sems[b]
            )

        for b in range(NBUF):
            chunk_dma(b, b)

        def chunk_body(g, b):
            c = g * NBUF + b
            pltpu.make_async_copy(
                feat_hbm.at[pl.ds(base + c * CH, CH)], buf_v.at[b], sems[b]
            ).wait()
            off = c * CH

            s_first = jnp.max(plsc.load_gather(
                ids_v, [jnp.full((L,), off, jnp.int32)]))
            s_last = jnp.max(plsc.load_gather(
                ids_v, [jnp.full((L,), off + CH - 1, jnp.int32)]))

            def seg_body(s, _):
                st = jnp.max(jnp.where(lane == s, lb, 0))
                en = jnp.max(jnp.where(lane == s, ub, 0))
                lo = jnp.clip(st - off, 0, CH)
                hi = jnp.clip(en - off, 0, CH)

                @pl.when(hi > lo)
                def _():
                    n = hi - lo
                    quads = lax.shift_right_logical(n, 2)

                    def row4(i, acc):
                        r = lo + i * 4
                        out = []
                        for j in range(DV):
                            sl = pl.ds(j * L, L)
                            x01 = buf_v[b, r, sl] + buf_v[b, r + 1, sl]
                            x23 = buf_v[b, r + 2, sl] + buf_v[b, r + 3, sl]
                            out.append(acc[j] + (x01 + x23))
                        return out

                    acc = lax.fori_loop(0, quads, row4, [zero] * DV)

                    def row(r, acc):
                        return [acc[j] + buf_v[b, r, pl.ds(j * L, L)]
                                for j in range(DV)]
                    acc = lax.fori_loop(lo + quads * 4, hi, row, acc)

                    for j in range(DV):
                        sl = pl.ds(j * L, L)
                        part_v[s, sl] = part_v[s, sl] + acc[j]
                return 0

            lax.fori_loop(s_first, s_last + 1, seg_body, 0)

            @pl.when(c + NBUF < NCHUNK)
            def _():
                chunk_dma(c + NBUF, b)

        def outer(g, _):
            for b in range(NBUF):
                chunk_body(g, b)
            return 0

        lax.fori_loop(0, NCHUNK // NBUF, outer, 0)

        pltpu.sync_copy(part_v, sums_hbm.at[wid])

    return k(features, ids32)


def _tc_partial_sums(features, ids32):
    ids3 = ids32.reshape(N // BR, 1, BR)

    def body(ids_ref, feat_ref, sums_ref, cnts_ref):
        i = pl.program_id(0)
        onehot = (ids_ref[0] == lax.broadcasted_iota(jnp.int32, (S, 1), 0)
                  ).astype(jnp.float32)
        psum = jax.lax.dot_general(
            onehot, feat_ref[...],
            dimension_numbers=(((1,), (0,)), ((), ())),
            preferred_element_type=jnp.float32)
        pcnt = jnp.sum(onehot, axis=1, keepdims=True)
        pcnt = jnp.broadcast_to(pcnt, (S, 128))

        @pl.when(i == 0)
        def _():
            sums_ref[...] = psum
            cnts_ref[...] = pcnt

        @pl.when(i > 0)
        def _():
            sums_ref[...] += psum
            cnts_ref[...] += pcnt

    return pl.pallas_call(
        body,
        grid=(NTCBLK,),
        in_specs=[
            pl.BlockSpec((1, 1, BR), lambda i: (i + NSC // BR, 0, 0)),
            pl.BlockSpec((BR, D), lambda i: (i + NSC // BR, 0)),
        ],
        out_specs=[
            pl.BlockSpec((S, D), lambda i: (0, 0)),
            pl.BlockSpec((S, 128), lambda i: (0, 0)),
        ],
        out_shape=[
            jax.ShapeDtypeStruct((S, D), jnp.float32),
            jax.ShapeDtypeStruct((S, 128), jnp.float32),
        ],
    )(ids3, features)


def _combine(psums, pcnts, tsums, tcnts):
    def body(ps_ref, pc_ref, ts_ref, tc_ref, out_ref):
        sums = jnp.sum(ps_ref[...], axis=0) + ts_ref[...]
        cnts = jnp.sum(pc_ref[...], axis=0) + tc_ref[:, 0]
        out_ref[...] = sums / jnp.maximum(cnts, 1.0)[:, None]

    return pl.pallas_call(
        body,
        out_shape=jax.ShapeDtypeStruct((S, D), jnp.float32),
    )(psums, pcnts, tsums, tcnts)


@jax.jit
def kernel(features, batch_ids):
    ids32 = batch_ids.astype(jnp.int32)
    psums, pcnts = _sc_partial_sums(features, ids32)
    tsums, tcnts = _tc_partial_sums(features, ids32)
    return _combine(psums, pcnts, tsums, tcnts)


# speedup vs baseline: 1.6761x; 1.1465x over previous
import functools

import jax
import jax.numpy as jnp
from jax import lax
from jax.experimental import pallas as pl
from jax.experimental.pallas import tpu as pltpu
from jax.experimental.pallas import tpu_sc as plsc

N = 32768
D = 512
S = 16
NC = 2
NS = 16
L = 16
NW = NC * NS

NSC = 8192
RW = NSC // NW
CH = 64
NCHUNK = RW // CH
NBUF = 2
DV = D // L

BR = 2048
NTCBLK = (N - NSC) // BR


def _sc_partial_sums(features, ids32):
    mesh = plsc.VectorSubcoreMesh(
        core_axis_name="c", subcore_axis_name="s", num_cores=NC, num_subcores=NS
    )

    @functools.partial(
        pl.kernel,
        out_type=(
            jax.ShapeDtypeStruct((NW, S, D), jnp.float32),
            jax.ShapeDtypeStruct((NW, S), jnp.float32),
        ),
        mesh=mesh,
        scratch_types=[
            pltpu.VMEM((RW,), jnp.int32),
            pltpu.VMEM((NBUF, CH, D), jnp.float32),
            pltpu.VMEM((S, D), jnp.float32),
            pltpu.VMEM((S,), jnp.float32),
        ] + [pltpu.SemaphoreType.DMA] * NBUF,
        compiler_params=pltpu.CompilerParams(needs_layout_passes=False),
    )
    def k(feat_hbm, ids_hbm, sums_hbm, cnts_hbm, ids_v, buf_v, part_v, cnt_v,
          *sems):
        wid = lax.axis_index("s") * NC + lax.axis_index("c")
        base = wid * RW

        pltpu.sync_copy(ids_hbm.at[pl.ds(base, RW)], ids_v)

        lane = lax.iota(jnp.int32, L)
        zero = jnp.zeros((L,), jnp.float32)

        def zbody(s, _):
            for j in range(DV):
                part_v[s, pl.ds(j * L, L)] = zero
            return 0
        lax.fori_loop(0, S, zbody, 0)

        def lower_bound(tgt):
            def step(_, lh):
                lo, hi = lh
                mid = lo + lax.shift_right_logical(hi - lo, 1)
                v = plsc.load_gather(ids_v, [jnp.minimum(mid, RW - 1)])
                go = mid < hi
                p = go & (v < tgt)
                lo = jnp.where(p, mid + 1, lo)
                hi = jnp.where(go & jnp.logical_not(p), mid, hi)
                return lo, hi
            lo0 = jnp.zeros((L,), jnp.int32)
            hi0 = jnp.full((L,), RW, jnp.int32)
            lo, _ = lax.fori_loop(0, 11, step, (lo0, hi0))
            return lo

        lb = lower_bound(lane)
        ub = lower_bound(lane + 1)

        cnt_v[...] = (ub - lb).astype(jnp.float32)
        pltpu.sync_copy(cnt_v, cnts_hbm.at[wid])

        def chunk_dma(c, b):
            return pltpu.async_copy(
                feat_hbm.at[pl.ds(base + c * CH, CH)], buf_v.at[b], sems[b]
            )

        for b in range(NBUF):
            chunk_dma(b, b)

        def chunk_body(g, b):
            c = g * NBUF + b
            pltpu.make_async_copy(
                feat_hbm.at[pl.ds(base + c * CH, CH)], buf_v.at[b], sems[b]
            ).wait()
            off = c * CH

            s_first = jnp.max(plsc.load_gather(
                ids_v, [jnp.full((L,), off, jnp.int32)]))
            s_last = jnp.max(plsc.load_gather(
                ids_v, [jnp.full((L,), off + CH - 1, jnp.int32)]))

            def seg_body(s, _):
                st = jnp.max(jnp.where(lane == s, lb, 0))
                en = jnp.max(jnp.where(lane == s, ub, 0))
                lo = jnp.clip(st - off, 0, CH)
                hi = jnp.clip(en - off, 0, CH)

                @pl.when(hi > lo)
                def _():
                    n = hi - lo
                    quads = lax.shift_right_logical(n, 2)

                    def row4(i, acc):
                        r = lo + i * 4
                        out = []
                        for j in range(DV):
                            sl = pl.ds(j * L, L)
                            x01 = buf_v[b, r, sl] + buf_v[b, r + 1, sl]
                            x23 = buf_v[b, r + 2, sl] + buf_v[b, r + 3, sl]
                            out.append(acc[j] + (x01 + x23))
                        return out

                    acc = lax.fori_loop(0, quads, row4, [zero] * DV)

                    def row(r, acc):
                        return [acc[j] + buf_v[b, r, pl.ds(j * L, L)]
                                for j in range(DV)]
                    acc = lax.fori_loop(lo + quads * 4, hi, row, acc)

                    for j in range(DV):
                        sl = pl.ds(j * L, L)
                        part_v[s, sl] = part_v[s, sl] + acc[j]
                return 0

            lax.fori_loop(s_first, s_last + 1, seg_body, 0)

            @pl.when(c + NBUF < NCHUNK)
            def _():
                chunk_dma(c + NBUF, b)

        def outer(g, _):
            for b in range(NBUF):
                chunk_body(g, b)
            return 0

        lax.fori_loop(0, NCHUNK // NBUF, outer, 0)

        pltpu.sync_copy(part_v, sums_hbm.at[wid])

    return k(features, ids32)


def _tc_partial_sums(features, ids32):
    ids3 = ids32.reshape(N // BR, 1, BR)

    def body(ids_ref, feat_ref, sums_ref, cnts_ref):
        i = pl.program_id(0)
        onehot = (ids_ref[0] == lax.broadcasted_iota(jnp.int32, (S, 1), 0)
                  ).astype(jnp.float32)
        psum = jax.lax.dot_general(
            onehot, feat_ref[...],
            dimension_numbers=(((1,), (0,)), ((), ())),
            preferred_element_type=jnp.float32)
        pcnt = jnp.sum(onehot, axis=1, keepdims=True)
        pcnt = jnp.broadcast_to(pcnt, (S, 128))

        @pl.when(i == 0)
        def _():
            sums_ref[...] = psum
            cnts_ref[...] = pcnt

        @pl.when(i > 0)
        def _():
            sums_ref[...] += psum
            cnts_ref[...] += pcnt

    return pl.pallas_call(
        body,
        grid=(NTCBLK,),
        in_specs=[
            pl.BlockSpec((1, 1, BR), lambda i: (i + NSC // BR, 0, 0)),
            pl.BlockSpec((BR, D), lambda i: (i + NSC // BR, 0)),
        ],
        out_specs=[
            pl.BlockSpec((S, D), lambda i: (0, 0)),
            pl.BlockSpec((S, 128), lambda i: (0, 0)),
        ],
        out_shape=[
            jax.ShapeDtypeStruct((S, D), jnp.float32),
            jax.ShapeDtypeStruct((S, 128), jnp.float32),
        ],
    )(ids3, features)


def _combine(psums, pcnts, tsums, tcnts):
    def body(ps_ref, pc_ref, ts_ref, tc_ref, out_ref):
        sums = jnp.sum(ps_ref[...], axis=0) + ts_ref[...]
        cnts = jnp.sum(pc_ref[...], axis=0) + tc_ref[:, 0]
        out_ref[...] = sums / jnp.maximum(cnts, 1.0)[:, None]

    return pl.pallas_call(
        body,
        out_shape=jax.ShapeDtypeStruct((S, D), jnp.float32),
    )(psums, pcnts, tsums, tcnts)


@jax.jit
def kernel(features, batch_ids):
    ids32 = batch_ids.astype(jnp.int32)
    psums, pcnts = _sc_partial_sums(features, ids32)
    tsums, tcnts = _tc_partial_sums(features, ids32)
    return _combine(psums, pcnts, tsums, tcnts)


# speedup vs baseline: 1.7022x; 1.0156x over previous
import functools

import jax
import jax.numpy as jnp
from jax import lax
from jax.experimental import pallas as pl
from jax.experimental.pallas import tpu as pltpu
from jax.experimental.pallas import tpu_sc as plsc

N = 32768
D = 512
S = 16
NC = 2
NS = 16
L = 16
NW = NC * NS

NSC = 4096
RW = NSC // NW
CH = 64
NCHUNK = RW // CH
NBUF = 2
DV = D // L

BR = 2048
NTCBLK = (N - NSC) // BR


def _sc_partial_sums(features, ids32):
    mesh = plsc.VectorSubcoreMesh(
        core_axis_name="c", subcore_axis_name="s", num_cores=NC, num_subcores=NS
    )

    @functools.partial(
        pl.kernel,
        out_type=(
            jax.ShapeDtypeStruct((NW, S, D), jnp.float32),
            jax.ShapeDtypeStruct((NW, S), jnp.float32),
        ),
        mesh=mesh,
        scratch_types=[
            pltpu.VMEM((RW,), jnp.int32),
            pltpu.VMEM((NBUF, CH, D), jnp.float32),
            pltpu.VMEM((S, D), jnp.float32),
            pltpu.VMEM((S,), jnp.float32),
        ] + [pltpu.SemaphoreType.DMA] * NBUF,
        compiler_params=pltpu.CompilerParams(needs_layout_passes=False),
    )
    def k(feat_hbm, ids_hbm, sums_hbm, cnts_hbm, ids_v, buf_v, part_v, cnt_v,
          *sems):
        wid = lax.axis_index("s") * NC + lax.axis_index("c")
        base = wid * RW

        pltpu.sync_copy(ids_hbm.at[pl.ds(base, RW)], ids_v)

        lane = lax.iota(jnp.int32, L)
        zero = jnp.zeros((L,), jnp.float32)

        def zbody(s, _):
            for j in range(DV):
                part_v[s, pl.ds(j * L, L)] = zero
            return 0
        lax.fori_loop(0, S, zbody, 0)

        def lower_bound(tgt):
            def step(_, lh):
                lo, hi = lh
                mid = lo + lax.shift_right_logical(hi - lo, 1)
                v = plsc.load_gather(ids_v, [jnp.minimum(mid, RW - 1)])
                go = mid < hi
                p = go & (v < tgt)
                lo = jnp.where(p, mid + 1, lo)
                hi = jnp.where(go & jnp.logical_not(p), mid, hi)
                return lo, hi
            lo0 = jnp.zeros((L,), jnp.int32)
            hi0 = jnp.full((L,), RW, jnp.int32)
            lo, _ = lax.fori_loop(0, 11, step, (lo0, hi0))
            return lo

        lb = lower_bound(lane)
        ub = lower_bound(lane + 1)

        cnt_v[...] = (ub - lb).astype(jnp.float32)
        pltpu.sync_copy(cnt_v, cnts_hbm.at[wid])

        def chunk_dma(c, b):
            return pltpu.async_copy(
                feat_hbm.at[pl.ds(base + c * CH, CH)], buf_v.at[b], sems[b]
            )

        for b in range(NBUF):
            chunk_dma(b, b)

        def chunk_body(g, b):
            c = g * NBUF + b
            pltpu.make_async_copy(
                feat_hbm.at[pl.ds(base + c * CH, CH)], buf_v.at[b], sems[b]
            ).wait()
            off = c * CH

            s_first = jnp.max(plsc.load_gather(
                ids_v, [jnp.full((L,), off, jnp.int32)]))
            s_last = jnp.max(plsc.load_gather(
                ids_v, [jnp.full((L,), off + CH - 1, jnp.int32)]))

            def seg_body(s, _):
                st = jnp.max(jnp.where(lane == s, lb, 0))
                en = jnp.max(jnp.where(lane == s, ub, 0))
                lo = jnp.clip(st - off, 0, CH)
                hi = jnp.clip(en - off, 0, CH)

                @pl.when(hi > lo)
                def _():
                    n = hi - lo
                    quads = lax.shift_right_logical(n, 2)

                    def row4(i, acc):
                        r = lo + i * 4
                        out = []
                        for j in range(DV):
                            sl = pl.ds(j * L, L)
                            x01 = buf_v[b, r, sl] + buf_v[b, r + 1, sl]
                            x23 = buf_v[b, r + 2, sl] + buf_v[b, r + 3, sl]
                            out.append(acc[j] + (x01 + x23))
                        return out

                    acc = lax.fori_loop(0, quads, row4, [zero] * DV)

                    def row(r, acc):
                        return [acc[j] + buf_v[b, r, pl.ds(j * L, L)]
                                for j in range(DV)]
                    acc = lax.fori_loop(lo + quads * 4, hi, row, acc)

                    for j in range(DV):
                        sl = pl.ds(j * L, L)
                        part_v[s, sl] = part_v[s, sl] + acc[j]
                return 0

            lax.fori_loop(s_first, s_last + 1, seg_body, 0)

            @pl.when(c + NBUF < NCHUNK)
            def _():
                chunk_dma(c + NBUF, b)

        def outer(g, _):
            for b in range(NBUF):
                chunk_body(g, b)
            return 0

        lax.fori_loop(0, NCHUNK // NBUF, outer, 0)

        pltpu.sync_copy(part_v, sums_hbm.at[wid])

    return k(features, ids32)


def _tc_partial_sums(features, ids32):
    ids3 = ids32.reshape(N // BR, 1, BR)

    def body(ids_ref, feat_ref, sums_ref, cnts_ref):
        i = pl.program_id(0)
        onehot = (ids_ref[0] == lax.broadcasted_iota(jnp.int32, (S, 1), 0)
                  ).astype(jnp.float32)
        psum = jax.lax.dot_general(
            onehot, feat_ref[...],
            dimension_numbers=(((1,), (0,)), ((), ())),
            preferred_element_type=jnp.float32)
        pcnt = jnp.sum(onehot, axis=1, keepdims=True)
        pcnt = jnp.broadcast_to(pcnt, (S, 128))

        @pl.when(i == 0)
        def _():
            sums_ref[...] = psum
            cnts_ref[...] = pcnt

        @pl.when(i > 0)
        def _():
            sums_ref[...] += psum
            cnts_ref[...] += pcnt

    return pl.pallas_call(
        body,
        grid=(NTCBLK,),
        in_specs=[
            pl.BlockSpec((1, 1, BR), lambda i: (i + NSC // BR, 0, 0)),
            pl.BlockSpec((BR, D), lambda i: (i + NSC // BR, 0)),
        ],
        out_specs=[
            pl.BlockSpec((S, D), lambda i: (0, 0)),
            pl.BlockSpec((S, 128), lambda i: (0, 0)),
        ],
        out_shape=[
            jax.ShapeDtypeStruct((S, D), jnp.float32),
            jax.ShapeDtypeStruct((S, 128), jnp.float32),
        ],
    )(ids3, features)


def _combine(psums, pcnts, tsums, tcnts):
    def body(ps_ref, pc_ref, ts_ref, tc_ref, out_ref):
        sums = jnp.sum(ps_ref[...], axis=0) + ts_ref[...]
        cnts = jnp.sum(pc_ref[...], axis=0) + tc_ref[:, 0]
        out_ref[...] = sums / jnp.maximum(cnts, 1.0)[:, None]

    return pl.pallas_call(
        body,
        out_shape=jax.ShapeDtypeStruct((S, D), jnp.float32),
    )(psums, pcnts, tsums, tcnts)


@jax.jit
def kernel(features, batch_ids):
    ids32 = batch_ids.astype(jnp.int32)
    psums, pcnts = _sc_partial_sums(features, ids32)
    tsums, tcnts = _tc_partial_sums(features, ids32)
    return _combine(psums, pcnts, tsums, tcnts)


# speedup vs baseline: 1.7569x; 1.0321x over previous
import functools

import jax
import jax.numpy as jnp
from jax import lax
from jax.experimental import pallas as pl
from jax.experimental.pallas import tpu as pltpu
from jax.experimental.pallas import tpu_sc as plsc

N = 32768
D = 512
S = 16
NC = 2
NS = 16
L = 16
NW = NC * NS

NSC = 4096
RW = NSC // NW
CH = 64
NCHUNK = RW // CH
NBUF = 2
DV = D // L

BR = 4096
NTCBLK = (N - NSC) // BR


def _sc_partial_sums(features, ids32):
    mesh = plsc.VectorSubcoreMesh(
        core_axis_name="c", subcore_axis_name="s", num_cores=NC, num_subcores=NS
    )

    @functools.partial(
        pl.kernel,
        out_type=(
            jax.ShapeDtypeStruct((NW, S, D), jnp.float32),
            jax.ShapeDtypeStruct((NW, S), jnp.float32),
        ),
        mesh=mesh,
        scratch_types=[
            pltpu.VMEM((RW,), jnp.int32),
            pltpu.VMEM((NBUF, CH, D), jnp.float32),
            pltpu.VMEM((S, D), jnp.float32),
            pltpu.VMEM((S,), jnp.float32),
        ] + [pltpu.SemaphoreType.DMA] * NBUF,
        compiler_params=pltpu.CompilerParams(needs_layout_passes=False),
    )
    def k(feat_hbm, ids_hbm, sums_hbm, cnts_hbm, ids_v, buf_v, part_v, cnt_v,
          *sems):
        wid = lax.axis_index("s") * NC + lax.axis_index("c")
        base = wid * RW

        pltpu.sync_copy(ids_hbm.at[pl.ds(base, RW)], ids_v)

        lane = lax.iota(jnp.int32, L)
        zero = jnp.zeros((L,), jnp.float32)

        def zbody(s, _):
            for j in range(DV):
                part_v[s, pl.ds(j * L, L)] = zero
            return 0
        lax.fori_loop(0, S, zbody, 0)

        def lower_bound(tgt):
            def step(_, lh):
                lo, hi = lh
                mid = lo + lax.shift_right_logical(hi - lo, 1)
                v = plsc.load_gather(ids_v, [jnp.minimum(mid, RW - 1)])
                go = mid < hi
                p = go & (v < tgt)
                lo = jnp.where(p, mid + 1, lo)
                hi = jnp.where(go & jnp.logical_not(p), mid, hi)
                return lo, hi
            lo0 = jnp.zeros((L,), jnp.int32)
            hi0 = jnp.full((L,), RW, jnp.int32)
            lo, _ = lax.fori_loop(0, 11, step, (lo0, hi0))
            return lo

        lb = lower_bound(lane)
        ub = lower_bound(lane + 1)

        cnt_v[...] = (ub - lb).astype(jnp.float32)
        pltpu.sync_copy(cnt_v, cnts_hbm.at[wid])

        def chunk_dma(c, b):
            return pltpu.async_copy(
                feat_hbm.at[pl.ds(base + c * CH, CH)], buf_v.at[b], sems[b]
            )

        for b in range(NBUF):
            chunk_dma(b, b)

        def chunk_body(g, b):
            c = g * NBUF + b
            pltpu.make_async_copy(
                feat_hbm.at[pl.ds(base + c * CH, CH)], buf_v.at[b], sems[b]
            ).wait()
            off = c * CH

            s_first = jnp.max(plsc.load_gather(
                ids_v, [jnp.full((L,), off, jnp.int32)]))
            s_last = jnp.max(plsc.load_gather(
                ids_v, [jnp.full((L,), off + CH - 1, jnp.int32)]))

            def seg_body(s, _):
                st = jnp.max(jnp.where(lane == s, lb, 0))
                en = jnp.max(jnp.where(lane == s, ub, 0))
                lo = jnp.clip(st - off, 0, CH)
                hi = jnp.clip(en - off, 0, CH)

                @pl.when(hi > lo)
                def _():
                    n = hi - lo
                    quads = lax.shift_right_logical(n, 2)

                    def row4(i, acc):
                        r = lo + i * 4
                        out = []
                        for j in range(DV):
                            sl = pl.ds(j * L, L)
                            x01 = buf_v[b, r, sl] + buf_v[b, r + 1, sl]
                            x23 = buf_v[b, r + 2, sl] + buf_v[b, r + 3, sl]
                            out.append(acc[j] + (x01 + x23))
                        return out

                    acc = lax.fori_loop(0, quads, row4, [zero] * DV)

                    def row(r, acc):
                        return [acc[j] + buf_v[b, r, pl.ds(j * L, L)]
                                for j in range(DV)]
                    acc = lax.fori_loop(lo + quads * 4, hi, row, acc)

                    for j in range(DV):
                        sl = pl.ds(j * L, L)
                        part_v[s, sl] = part_v[s, sl] + acc[j]
                return 0

            lax.fori_loop(s_first, s_last + 1, seg_body, 0)

            @pl.when(c + NBUF < NCHUNK)
            def _():
                chunk_dma(c + NBUF, b)

        def outer(g, _):
            for b in range(NBUF):
                chunk_body(g, b)
            return 0

        lax.fori_loop(0, NCHUNK // NBUF, outer, 0)

        pltpu.sync_copy(part_v, sums_hbm.at[wid])

    return k(features, ids32)


def _tc_partial_sums(features, ids32):
    ids3 = ids32.reshape(N // BR, 1, BR)

    def body(ids_ref, feat_ref, sums_ref, cnts_ref):
        i = pl.program_id(0)
        onehot = (ids_ref[0] == lax.broadcasted_iota(jnp.int32, (S, 1), 0)
                  ).astype(jnp.float32)
        psum = jax.lax.dot_general(
            onehot, feat_ref[...],
            dimension_numbers=(((1,), (0,)), ((), ())),
            preferred_element_type=jnp.float32)
        pcnt = jnp.sum(onehot, axis=1, keepdims=True)
        pcnt = jnp.broadcast_to(pcnt, (S, 128))

        @pl.when(i == 0)
        def _():
            sums_ref[...] = psum
            cnts_ref[...] = pcnt

        @pl.when(i > 0)
        def _():
            sums_ref[...] += psum
            cnts_ref[...] += pcnt

    return pl.pallas_call(
        body,
        grid=(NTCBLK,),
        in_specs=[
            pl.BlockSpec((1, 1, BR), lambda i: (i + NSC // BR, 0, 0)),
            pl.BlockSpec((BR, D), lambda i: (i + NSC // BR, 0)),
        ],
        out_specs=[
            pl.BlockSpec((S, D), lambda i: (0, 0)),
            pl.BlockSpec((S, 128), lambda i: (0, 0)),
        ],
        out_shape=[
            jax.ShapeDtypeStruct((S, D), jnp.float32),
            jax.ShapeDtypeStruct((S, 128), jnp.float32),
        ],
    )(ids3, features)


def _combine(psums, pcnts, tsums, tcnts):
    def body(ps_ref, pc_ref, ts_ref, tc_ref, out_ref):
        sums = jnp.sum(ps_ref[...], axis=0) + ts_ref[...]
        cnts = jnp.sum(pc_ref[...], axis=0) + tc_ref[:, 0]
        out_ref[...] = sums / jnp.maximum(cnts, 1.0)[:, None]

    return pl.pallas_call(
        body,
        out_shape=jax.ShapeDtypeStruct((S, D), jnp.float32),
    )(psums, pcnts, tsums, tcnts)


@jax.jit
def kernel(features, batch_ids):
    ids32 = batch_ids.astype(jnp.int32)
    psums, pcnts = _sc_partial_sums(features, ids32)
    tsums, tcnts = _tc_partial_sums(features, ids32)
    return _combine(psums, pcnts, tsums, tcnts)
